# baseline (speedup 1.0000x reference)
import functools

import jax
import jax.numpy as jnp
from jax import lax
from jax.experimental import pallas as pl
from jax.experimental.pallas import tpu as pltpu
from jax.experimental.pallas import tpu_sc as plsc

_N = 10000
_D = 128
_E = 320000

_NC = 2
_NS = 16
_NW = _NC * _NS
_EPW = _E // _NW
_K = 125
_NCHUNK = _EPW // _K
_NP = 10240
_RPT = _NP // _NS
_ZR = 32
_DEGN = 10240
_DPT = _DEGN // _NS

_mesh = plsc.VectorSubcoreMesh(core_axis_name="c", subcore_axis_name="s")


def _sc_deg_body(ei_hbm, ones_hbm, zer_hbm, out_hbm, deg_sh, didx, onev):
    cid = lax.axis_index("c")
    sid = lax.axis_index("s")
    wid = sid * _NC + cid

    pltpu.sync_copy(ei_hbm.at[1, wid], didx)
    pltpu.sync_copy(ones_hbm, onev)
    pltpu.sync_copy(zer_hbm, deg_sh.at[pl.ds(sid * _DPT, _DPT)])
    plsc.subcore_barrier()

    def chunk(i, c):
        pltpu.sync_copy(onev, deg_sh.at[didx.at[i]], add=True)
        return c

    lax.fori_loop(0, _NCHUNK, chunk, 0)
    plsc.subcore_barrier()
    pltpu.sync_copy(deg_sh.at[pl.ds(sid * _DPT, _DPT)],
                    out_hbm.at[pl.ds(cid * _DEGN + sid * _DPT, _DPT)])


_sc_deg = pl.kernel(
    _sc_deg_body,
    out_type=jax.ShapeDtypeStruct((_NC * _DEGN, 1), jnp.float32),
    mesh=_mesh,
    scratch_types=[
        pltpu.VMEM_SHARED((_DEGN, 1), jnp.float32),
        pltpu.VMEM((_NCHUNK, _K), jnp.int32),
        pltpu.VMEM((_K, 1), jnp.float32),
    ],
)


def _sc_agg_body(y_hbm, ei_hbm, out_hbm, acc_sh, ib0, ib1, ib2, ib3,
                 rows0, rows1, zbuf, gsem0, gsem1, ssem0, ssem1,
                 isem0, isem1, isem2, isem3):
    cid = lax.axis_index("c")
    sid = lax.axis_index("s")
    wid = sid * _NC + cid
    ib = [ib0, ib1, ib2, ib3]
    isem = [isem0, isem1, isem2, isem3]
    rows = [rows0, rows1]
    gsem = [gsem0, gsem1]
    ssem = [ssem0, ssem1]

    def zrow(i, c):
        def zcol(j, c2):
            zbuf[i, pl.ds(j * 16, 16)] = jnp.zeros((16,), jnp.float32)
            return c2
        return lax.fori_loop(0, _D // 16, zcol, c)

    pltpu.async_copy(ei_hbm.at[0, wid, 0], ib0.at[0], isem0)
    pltpu.async_copy(ei_hbm.at[1, wid, 0], ib0.at[1], isem0)
    pltpu.async_copy(ei_hbm.at[0, wid, 1], ib1.at[0], isem1)
    pltpu.async_copy(ei_hbm.at[1, wid, 1], ib1.at[1], isem1)
    lax.fori_loop(0, _ZR, zrow, 0)
    for z in range(_RPT // _ZR):
        pltpu.sync_copy(zbuf, acc_sh.at[pl.ds(sid * _RPT + z * _ZR, _ZR)])
    plsc.subcore_barrier()

    def slot(j, s, skip_ssem=False, skip_idx=False, skip_prev=False):
        b, q = s % 2, s % 4
        if not skip_ssem:
            pltpu.make_async_copy(rows[b], acc_sh.at[ib[q].at[1]],
                                  ssem[b]).wait()
        pltpu.make_async_copy(ei_hbm.at[0, wid, 0], ib[q].at[0],
                              isem[q]).wait()
        pltpu.make_async_copy(ei_hbm.at[1, wid, 0], ib[q].at[1],
                              isem[q]).wait()
        pltpu.async_copy(y_hbm.at[ib[q].at[0]], rows[b], gsem[b])
        if not skip_idx:
            q2 = (q + 2) % 4
            pltpu.async_copy(ei_hbm.at[0, wid, j + 2], ib[q2].at[0],
                             isem[q2])
            pltpu.async_copy(ei_hbm.at[1, wid, j + 2], ib[q2].at[1],
                             isem[q2])
        if not skip_prev:
            pb, pq = (s - 1) % 2, (s - 1) % 4
            pltpu.make_async_copy(y_hbm.at[ib[pq].at[0]], rows[pb],
                                  gsem[pb]).wait()
            pltpu.async_copy(rows[pb], acc_sh.at[ib[pq].at[1]], ssem[pb],
                             add=True)

    slot(0, 0, skip_ssem=True, skip_prev=True)
    slot(1, 1, skip_ssem=True)
    slot(2, 2)
    slot(3, 3)

    def body(t, c):
        for s in range(4):
            slot(4 * t + s, s)
        return c

    lax.fori_loop(1, _NCHUNK // 4 - 1, body, 0)

    tl = _NCHUNK - 4
    slot(tl, 0)
    slot(tl + 1, 1)
    slot(tl + 2, 2, skip_idx=True)
    slot(tl + 3, 3, skip_idx=True)

    pltpu.make_async_copy(y_hbm.at[ib3.at[0]], rows1, gsem1).wait()
    pltpu.async_copy(rows1, acc_sh.at[ib3.at[1]], ssem1, add=True)
    pltpu.make_async_copy(rows0, acc_sh.at[ib2.at[1]], ssem0).wait()
    pltpu.make_async_copy(rows1, acc_sh.at[ib3.at[1]], ssem1).wait()

    plsc.subcore_barrier()
    pltpu.sync_copy(acc_sh.at[pl.ds(sid * _RPT, _RPT)],
                    out_hbm.at[pl.ds(cid * _NP + sid * _RPT, _RPT)])


_sc_agg = pl.kernel(
    _sc_agg_body,
    out_type=jax.ShapeDtypeStruct((_NC * _NP, _D), jnp.float32),
    mesh=_mesh,
    scratch_types=[
        pltpu.VMEM_SHARED((_NP, _D), jnp.float32),
        pltpu.VMEM((2, _K), jnp.int32),
        pltpu.VMEM((2, _K), jnp.int32),
        pltpu.VMEM((2, _K), jnp.int32),
        pltpu.VMEM((2, _K), jnp.int32),
        pltpu.VMEM((_K, _D), jnp.float32),
        pltpu.VMEM((_K, _D), jnp.float32),
        pltpu.VMEM((_ZR, _D), jnp.float32),
        pltpu.SemaphoreType.DMA,
        pltpu.SemaphoreType.DMA,
        pltpu.SemaphoreType.DMA,
        pltpu.SemaphoreType.DMA,
        pltpu.SemaphoreType.DMA,
        pltpu.SemaphoreType.DMA,
        pltpu.SemaphoreType.DMA,
        pltpu.SemaphoreType.DMA,
    ],
)


_BN = 2048
_GRID = _NP // _BN


def _tc_prep_body(x_ref, w_ref, dg_ref, d1_ref, y_ref):
    d0_ref = dg_ref
    dinv = lax.rsqrt(d0_ref[...] + d1_ref[...] + 1.0)
    y_ref[...] = jnp.dot(x_ref[...], w_ref[...],
                         preferred_element_type=jnp.float32) * dinv


_tc_prep = pl.pallas_call(
    _tc_prep_body,
    grid=(_GRID,),
    in_specs=[
        pl.BlockSpec((_BN, _D), lambda i: (i, 0)),
        pl.BlockSpec((_D, _D), lambda i: (0, 0)),
        pl.BlockSpec((_BN, 1), lambda i: (i, 0)),
        pl.BlockSpec((_BN, 1), lambda i: (i + _GRID, 0)),
    ],
    out_specs=pl.BlockSpec((_BN, _D), lambda i: (i, 0)),
    out_shape=jax.ShapeDtypeStruct((_NP, _D), jnp.float32),
)


def _tc_mid_body(a0_ref, a1_ref, y_ref, d0_ref, d1_ref, b_ref, w_ref,
                 o_ref):
    dinv = lax.rsqrt(d0_ref[...] + d1_ref[...] + 1.0)
    h = dinv * (a0_ref[...] + a1_ref[...] + y_ref[...]) + b_ref[...]
    h = jnp.maximum(h, 0.0)
    o_ref[...] = jnp.dot(h, w_ref[...],
                         preferred_element_type=jnp.float32) * dinv


_tc_mid = pl.pallas_call(
    _tc_mid_body,
    grid=(_GRID,),
    in_specs=[
        pl.BlockSpec((_BN, _D), lambda i: (i, 0)),
        pl.BlockSpec((_BN, _D), lambda i: (i + _GRID, 0)),
        pl.BlockSpec((_BN, _D), lambda i: (i, 0)),
        pl.BlockSpec((_BN, 1), lambda i: (i, 0)),
        pl.BlockSpec((_BN, 1), lambda i: (i + _GRID, 0)),
        pl.BlockSpec((1, _D), lambda i: (0, 0)),
        pl.BlockSpec((_D, _D), lambda i: (0, 0)),
    ],
    out_specs=pl.BlockSpec((_BN, _D), lambda i: (i, 0)),
    out_shape=jax.ShapeDtypeStruct((_NP, _D), jnp.float32),
)


def _tc_out_body(a0_ref, a1_ref, y_ref, d0_ref, d1_ref, b_ref, o_ref):
    dinv = lax.rsqrt(d0_ref[...] + d1_ref[...] + 1.0)
    h = dinv * (a0_ref[...] + a1_ref[...] + y_ref[...]) + b_ref[...]
    o_ref[...] = jnp.maximum(h, 0.0)


_tc_out = pl.pallas_call(
    _tc_out_body,
    grid=(_GRID,),
    in_specs=[
        pl.BlockSpec((_BN, _D), lambda i: (i, 0)),
        pl.BlockSpec((_BN, _D), lambda i: (i + _GRID, 0)),
        pl.BlockSpec((_BN, _D), lambda i: (i, 0)),
        pl.BlockSpec((_BN, 1), lambda i: (i, 0)),
        pl.BlockSpec((_BN, 1), lambda i: (i + _GRID, 0)),
        pl.BlockSpec((1, _D), lambda i: (0, 0)),
    ],
    out_specs=pl.BlockSpec((_BN, _D), lambda i: (i, 0)),
    out_shape=jax.ShapeDtypeStruct((_NP, _D), jnp.float32),
)


@jax.jit
def kernel(x, edge_index, W1, b1, W2, b2):
    ei = edge_index.reshape(2, _NW, _NCHUNK, _K)
    xp = jnp.pad(x, ((0, _NP - _N), (0, 0)))
    degp = _sc_deg(ei, jnp.ones((_K, 1), jnp.float32),
                   jnp.zeros((_DPT, 1), jnp.float32))
    y1 = _tc_prep(xp, W1, degp, degp)
    accp1 = _sc_agg(y1, ei)
    y2 = _tc_mid(accp1, accp1, y1, degp, degp, b1.reshape(1, _D), W2)
    accp2 = _sc_agg(y2, ei)
    out = _tc_out(accp2, accp2, y2, degp, degp, b2.reshape(1, _D))
    return out[:_N]


# speedup vs baseline: 1.6194x; 1.6194x over previous
import functools

import jax
import jax.numpy as jnp
from jax import lax
from jax.experimental import pallas as pl
from jax.experimental.pallas import tpu as pltpu
from jax.experimental.pallas import tpu_sc as plsc

_N = 10000
_D = 128
_E = 320000

_NC = 2
_NS = 16
_NW = _NC * _NS
_EPW = _E // _NW
_K = 125
_NCHUNK = _EPW // _K
_NP = 10240
_RPT = _NP // _NS
_ZR = 32
_DEGN = 10240
_DPT = _DEGN // _NS

_mesh = plsc.VectorSubcoreMesh(core_axis_name="c", subcore_axis_name="s")


def _sc_deg_body(ei_hbm, out_hbm, deg_sh, didx, onesb, zbuf):
    cid = lax.axis_index("c")
    sid = lax.axis_index("s")
    wid = sid * _NC + cid

    def fill_z(i, c):
        zbuf[pl.ds(i * 16, 16)] = jnp.zeros((16,), jnp.float32)
        return c

    lax.fori_loop(0, _DPT // 16, fill_z, 0)

    def fill_one(i, c):
        onesb[pl.ds(i * 16, 16)] = jnp.ones((16,), jnp.float32)
        return c

    lax.fori_loop(0, 8, fill_one, 0)

    pltpu.sync_copy(ei_hbm.at[1, wid], didx)
    pltpu.sync_copy(zbuf, deg_sh.at[pl.ds(sid * _DPT, _DPT)])
    plsc.subcore_barrier()

    def chunk(i, c):
        pltpu.sync_copy(onesb.at[pl.ds(0, _K)], deg_sh.at[didx.at[i]],
                        add=True)
        return c

    lax.fori_loop(0, _NCHUNK, chunk, 0)
    plsc.subcore_barrier()
    pltpu.sync_copy(deg_sh.at[pl.ds(sid * _DPT, _DPT)],
                    out_hbm.at[pl.ds(cid * _DEGN + sid * _DPT, _DPT)])


_sc_deg = pl.kernel(
    _sc_deg_body,
    out_type=jax.ShapeDtypeStruct((_NC * _DEGN,), jnp.float32),
    mesh=_mesh,
    scratch_types=[
        pltpu.VMEM_SHARED((_DEGN,), jnp.float32),
        pltpu.VMEM((_NCHUNK, _K), jnp.int32),
        pltpu.VMEM((128,), jnp.float32),
        pltpu.VMEM((_DPT,), jnp.float32),
    ],
)


def _sc_agg_body(y_hbm, ei_hbm, out_hbm, acc_sh, ib0, ib1, ib2, ib3,
                 rows0, rows1, zbuf, gsem0, gsem1, ssem0, ssem1,
                 isem0, isem1, isem2, isem3):
    cid = lax.axis_index("c")
    sid = lax.axis_index("s")
    wid = sid * _NC + cid
    ib = [ib0, ib1, ib2, ib3]
    isem = [isem0, isem1, isem2, isem3]
    rows = [rows0, rows1]
    gsem = [gsem0, gsem1]
    ssem = [ssem0, ssem1]

    def zrow(i, c):
        def zcol(j, c2):
            zbuf[i, pl.ds(j * 16, 16)] = jnp.zeros((16,), jnp.float32)
            return c2
        return lax.fori_loop(0, _D // 16, zcol, c)

    pltpu.async_copy(ei_hbm.at[0, wid, 0], ib0.at[0], isem0)
    pltpu.async_copy(ei_hbm.at[1, wid, 0], ib0.at[1], isem0)
    pltpu.async_copy(ei_hbm.at[0, wid, 1], ib1.at[0], isem1)
    pltpu.async_copy(ei_hbm.at[1, wid, 1], ib1.at[1], isem1)
    lax.fori_loop(0, _ZR, zrow, 0)
    for z in range(_RPT // _ZR):
        pltpu.sync_copy(zbuf, acc_sh.at[pl.ds(sid * _RPT + z * _ZR, _ZR)])
    plsc.subcore_barrier()

    def slot(j, s, skip_ssem=False, skip_idx=False, skip_prev=False):
        b, q = s % 2, s % 4
        if not skip_ssem:
            pltpu.make_async_copy(rows[b], acc_sh.at[ib[q].at[1]],
                                  ssem[b]).wait()
        pltpu.make_async_copy(ei_hbm.at[0, wid, 0], ib[q].at[0],
                              isem[q]).wait()
        pltpu.make_async_copy(ei_hbm.at[1, wid, 0], ib[q].at[1],
                              isem[q]).wait()
        pltpu.async_copy(y_hbm.at[ib[q].at[0]], rows[b], gsem[b])
        if not skip_idx:
            q2 = (q + 2) % 4
            pltpu.async_copy(ei_hbm.at[0, wid, j + 2], ib[q2].at[0],
                             isem[q2])
            pltpu.async_copy(ei_hbm.at[1, wid, j + 2], ib[q2].at[1],
                             isem[q2])
        if not skip_prev:
            pb, pq = (s - 1) % 2, (s - 1) % 4
            pltpu.make_async_copy(y_hbm.at[ib[pq].at[0]], rows[pb],
                                  gsem[pb]).wait()
            pltpu.async_copy(rows[pb], acc_sh.at[ib[pq].at[1]], ssem[pb],
                             add=True)

    slot(0, 0, skip_ssem=True, skip_prev=True)
    slot(1, 1, skip_ssem=True)
    slot(2, 2)
    slot(3, 3)

    def body(t, c):
        for s in range(4):
            slot(4 * t + s, s)
        return c

    lax.fori_loop(1, _NCHUNK // 4 - 1, body, 0)

    tl = _NCHUNK - 4
    slot(tl, 0)
    slot(tl + 1, 1)
    slot(tl + 2, 2, skip_idx=True)
    slot(tl + 3, 3, skip_idx=True)

    pltpu.make_async_copy(y_hbm.at[ib3.at[0]], rows1, gsem1).wait()
    pltpu.async_copy(rows1, acc_sh.at[ib3.at[1]], ssem1, add=True)
    pltpu.make_async_copy(rows0, acc_sh.at[ib2.at[1]], ssem0).wait()
    pltpu.make_async_copy(rows1, acc_sh.at[ib3.at[1]], ssem1).wait()

    plsc.subcore_barrier()
    pltpu.sync_copy(acc_sh.at[pl.ds(sid * _RPT, _RPT)],
                    out_hbm.at[pl.ds(cid * _NP + sid * _RPT, _RPT)])


_sc_agg = pl.kernel(
    _sc_agg_body,
    out_type=jax.ShapeDtypeStruct((_NC * _NP, _D), jnp.float32),
    mesh=_mesh,
    scratch_types=[
        pltpu.VMEM_SHARED((_NP, _D), jnp.float32),
        pltpu.VMEM((2, _K), jnp.int32),
        pltpu.VMEM((2, _K), jnp.int32),
        pltpu.VMEM((2, _K), jnp.int32),
        pltpu.VMEM((2, _K), jnp.int32),
        pltpu.VMEM((_K, _D), jnp.float32),
        pltpu.VMEM((_K, _D), jnp.float32),
        pltpu.VMEM((_ZR, _D), jnp.float32),
        pltpu.SemaphoreType.DMA,
        pltpu.SemaphoreType.DMA,
        pltpu.SemaphoreType.DMA,
        pltpu.SemaphoreType.DMA,
        pltpu.SemaphoreType.DMA,
        pltpu.SemaphoreType.DMA,
        pltpu.SemaphoreType.DMA,
        pltpu.SemaphoreType.DMA,
    ],
)


_BN = 2048
_GRID = _NP // _BN


def _tc_prep_body(x_ref, w_ref, dg_ref, d1_ref, y_ref):
    d0_ref = dg_ref
    dinv = lax.rsqrt(d0_ref[...] + d1_ref[...] + 1.0)
    y_ref[...] = jnp.dot(x_ref[...], w_ref[...],
                         preferred_element_type=jnp.float32) * dinv


_tc_prep = pl.pallas_call(
    _tc_prep_body,
    grid=(_GRID,),
    in_specs=[
        pl.BlockSpec((_BN, _D), lambda i: (i, 0)),
        pl.BlockSpec((_D, _D), lambda i: (0, 0)),
        pl.BlockSpec((_BN, 1), lambda i: (i, 0)),
        pl.BlockSpec((_BN, 1), lambda i: (i + _GRID, 0)),
    ],
    out_specs=pl.BlockSpec((_BN, _D), lambda i: (i, 0)),
    out_shape=jax.ShapeDtypeStruct((_NP, _D), jnp.float32),
)


def _tc_mid_body(a0_ref, a1_ref, y_ref, d0_ref, d1_ref, b_ref, w_ref,
                 o_ref):
    dinv = lax.rsqrt(d0_ref[...] + d1_ref[...] + 1.0)
    h = dinv * (a0_ref[...] + a1_ref[...] + y_ref[...]) + b_ref[...]
    h = jnp.maximum(h, 0.0)
    o_ref[...] = jnp.dot(h, w_ref[...],
                         preferred_element_type=jnp.float32) * dinv


_tc_mid = pl.pallas_call(
    _tc_mid_body,
    grid=(_GRID,),
    in_specs=[
        pl.BlockSpec((_BN, _D), lambda i: (i, 0)),
        pl.BlockSpec((_BN, _D), lambda i: (i + _GRID, 0)),
        pl.BlockSpec((_BN, _D), lambda i: (i, 0)),
        pl.BlockSpec((_BN, 1), lambda i: (i, 0)),
        pl.BlockSpec((_BN, 1), lambda i: (i + _GRID, 0)),
        pl.BlockSpec((1, _D), lambda i: (0, 0)),
        pl.BlockSpec((_D, _D), lambda i: (0, 0)),
    ],
    out_specs=pl.BlockSpec((_BN, _D), lambda i: (i, 0)),
    out_shape=jax.ShapeDtypeStruct((_NP, _D), jnp.float32),
)


def _tc_out_body(a0_ref, a1_ref, y_ref, d0_ref, d1_ref, b_ref, o_ref):
    dinv = lax.rsqrt(d0_ref[...] + d1_ref[...] + 1.0)
    h = dinv * (a0_ref[...] + a1_ref[...] + y_ref[...]) + b_ref[...]
    o_ref[...] = jnp.maximum(h, 0.0)


_tc_out = pl.pallas_call(
    _tc_out_body,
    grid=(_GRID,),
    in_specs=[
        pl.BlockSpec((_BN, _D), lambda i: (i, 0)),
        pl.BlockSpec((_BN, _D), lambda i: (i + _GRID, 0)),
        pl.BlockSpec((_BN, _D), lambda i: (i, 0)),
        pl.BlockSpec((_BN, 1), lambda i: (i, 0)),
        pl.BlockSpec((_BN, 1), lambda i: (i + _GRID, 0)),
        pl.BlockSpec((1, _D), lambda i: (0, 0)),
    ],
    out_specs=pl.BlockSpec((_BN, _D), lambda i: (i, 0)),
    out_shape=jax.ShapeDtypeStruct((_NP, _D), jnp.float32),
)


@jax.jit
def kernel(x, edge_index, W1, b1, W2, b2):
    ei = edge_index.reshape(2, _NW, _NCHUNK, _K)
    xp = jnp.pad(x, ((0, _NP - _N), (0, 0)))
    degp = _sc_deg(ei).reshape(_NC * _DEGN, 1)
    y1 = _tc_prep(xp, W1, degp, degp)
    accp1 = _sc_agg(y1, ei)
    y2 = _tc_mid(accp1, accp1, y1, degp, degp, b1.reshape(1, _D), W2)
    accp2 = _sc_agg(y2, ei)
    out = _tc_out(accp2, accp2, y2, degp, degp, b2.reshape(1, _D))
    return out[:_N]


# speedup vs baseline: 1.6211x; 1.0011x over previous
import jax
import jax.numpy as jnp
from jax import lax
from jax.experimental import pallas as pl
from jax.experimental.pallas import tpu as pltpu
from jax.experimental.pallas import tpu_sc as plsc

_N = 10000
_D = 128
_E = 320000

_NC = 2
_NS = 16
_NW = _NC * _NS
_EPW = _E // _NW
_K = 125
_NCHUNK = _EPW // _K
_NP = 10240
_RPT = _NP // _NS
_ZR = 32
_DEGN = 10240
_DPT = _DEGN // _NS

_mesh = plsc.VectorSubcoreMesh(core_axis_name="c", subcore_axis_name="s")


def _sc_deg_body(ei_hbm, out_hbm, deg_sh, didx, onesb, zbuf):
    cid = lax.axis_index("c")
    sid = lax.axis_index("s")
    wid = sid * _NC + cid

    def fill_z(i, c):
        zbuf[pl.ds(i * 16, 16)] = jnp.zeros((16,), jnp.float32)
        return c

    lax.fori_loop(0, _DPT // 16, fill_z, 0)

    def fill_one(i, c):
        onesb[pl.ds(i * 16, 16)] = jnp.ones((16,), jnp.float32)
        return c

    lax.fori_loop(0, 8, fill_one, 0)

    pltpu.sync_copy(ei_hbm.at[1, wid], didx)
    pltpu.sync_copy(zbuf, deg_sh.at[pl.ds(sid * _DPT, _DPT)])
    plsc.subcore_barrier()

    def chunk(i, c):
        pltpu.sync_copy(onesb.at[pl.ds(0, _K)], deg_sh.at[didx.at[i]],
                        add=True)
        return c

    lax.fori_loop(0, _NCHUNK, chunk, 0)
    plsc.subcore_barrier()
    pltpu.sync_copy(deg_sh.at[pl.ds(sid * _DPT, _DPT)],
                    out_hbm.at[pl.ds(cid * _DEGN + sid * _DPT, _DPT)])


_sc_deg = pl.kernel(
    _sc_deg_body,
    out_type=jax.ShapeDtypeStruct((_NC * _DEGN,), jnp.float32),
    mesh=_mesh,
    scratch_types=[
        pltpu.VMEM_SHARED((_DEGN,), jnp.float32),
        pltpu.VMEM((_NCHUNK, _K), jnp.int32),
        pltpu.VMEM((128,), jnp.float32),
        pltpu.VMEM((_DPT,), jnp.float32),
    ],
)


def _sc_agg_body(y_hbm, ei_hbm, out_hbm, acc_sh, ib0, ib1, ib2, ib3,
                 rows0, rows1, zbuf, gsem0, gsem1, ssem0, ssem1,
                 isem0, isem1, isem2, isem3):
    cid = lax.axis_index("c")
    sid = lax.axis_index("s")
    wid = sid * _NC + cid
    ib = [ib0, ib1, ib2, ib3]
    isem = [isem0, isem1, isem2, isem3]
    rows = [rows0, rows1]
    gsem = [gsem0, gsem1]
    ssem = [ssem0, ssem1]

    def zrow(i, c):
        def zcol(j, c2):
            zbuf[i, pl.ds(j * 16, 16)] = jnp.zeros((16,), jnp.float32)
            return c2
        return lax.fori_loop(0, _D // 16, zcol, c)

    pltpu.async_copy(ei_hbm.at[0, wid, 0], ib0.at[0], isem0)
    pltpu.async_copy(ei_hbm.at[1, wid, 0], ib0.at[1], isem0)
    pltpu.async_copy(ei_hbm.at[0, wid, 1], ib1.at[0], isem1)
    pltpu.async_copy(ei_hbm.at[1, wid, 1], ib1.at[1], isem1)
    lax.fori_loop(0, _ZR, zrow, 0)
    for z in range(_RPT // _ZR):
        pltpu.sync_copy(zbuf, acc_sh.at[pl.ds(sid * _RPT + z * _ZR, _ZR)])
    plsc.subcore_barrier()

    def slot(j, s, skip_ssem=False, skip_idx=False, skip_prev=False):
        b, q = s % 2, s % 4
        if not skip_ssem:
            pltpu.make_async_copy(rows[b], acc_sh.at[ib[q].at[1]],
                                  ssem[b]).wait()
        pltpu.make_async_copy(ei_hbm.at[0, wid, 0], ib[q].at[0],
                              isem[q]).wait()
        pltpu.make_async_copy(ei_hbm.at[1, wid, 0], ib[q].at[1],
                              isem[q]).wait()
        pltpu.async_copy(y_hbm.at[ib[q].at[0]], rows[b], gsem[b])
        if not skip_idx:
            q2 = (q + 2) % 4
            pltpu.async_copy(ei_hbm.at[0, wid, j + 2], ib[q2].at[0],
                             isem[q2])
            pltpu.async_copy(ei_hbm.at[1, wid, j + 2], ib[q2].at[1],
                             isem[q2])
        if not skip_prev:
            pb, pq = (s - 1) % 2, (s - 1) % 4
            pltpu.make_async_copy(y_hbm.at[ib[pq].at[0]], rows[pb],
                                  gsem[pb]).wait()
            pltpu.async_copy(rows[pb], acc_sh.at[ib[pq].at[1]], ssem[pb],
                             add=True)

    slot(0, 0, skip_ssem=True, skip_prev=True)
    slot(1, 1, skip_ssem=True)
    slot(2, 2)
    slot(3, 3)

    def body(t, c):
        for s in range(4):
            slot(4 * t + s, s)
        return c

    lax.fori_loop(1, _NCHUNK // 4 - 1, body, 0)

    tl = _NCHUNK - 4
    slot(tl, 0)
    slot(tl + 1, 1)
    slot(tl + 2, 2, skip_idx=True)
    slot(tl + 3, 3, skip_idx=True)

    pltpu.make_async_copy(y_hbm.at[ib3.at[0]], rows1, gsem1).wait()
    pltpu.async_copy(rows1, acc_sh.at[ib3.at[1]], ssem1, add=True)
    pltpu.make_async_copy(rows0, acc_sh.at[ib2.at[1]], ssem0).wait()
    pltpu.make_async_copy(rows1, acc_sh.at[ib3.at[1]], ssem1).wait()

    plsc.subcore_barrier()
    pltpu.sync_copy(acc_sh.at[pl.ds(sid * _RPT, _RPT)],
                    out_hbm.at[pl.ds(cid * _NP + sid * _RPT, _RPT)])


_sc_agg = pl.kernel(
    _sc_agg_body,
    out_type=jax.ShapeDtypeStruct((_NC * _NP, _D), jnp.float32),
    mesh=_mesh,
    scratch_types=[
        pltpu.VMEM_SHARED((_NP, _D), jnp.float32),
        pltpu.VMEM((2, _K), jnp.int32),
        pltpu.VMEM((2, _K), jnp.int32),
        pltpu.VMEM((2, _K), jnp.int32),
        pltpu.VMEM((2, _K), jnp.int32),
        pltpu.VMEM((_K, _D), jnp.float32),
        pltpu.VMEM((_K, _D), jnp.float32),
        pltpu.VMEM((_ZR, _D), jnp.float32),
        pltpu.SemaphoreType.DMA,
        pltpu.SemaphoreType.DMA,
        pltpu.SemaphoreType.DMA,
        pltpu.SemaphoreType.DMA,
        pltpu.SemaphoreType.DMA,
        pltpu.SemaphoreType.DMA,
        pltpu.SemaphoreType.DMA,
        pltpu.SemaphoreType.DMA,
    ],
)


_BN = 2048
_GRID = _NP // _BN


def _tc_prep_body(x_ref, w_ref, dg_ref, d1_ref, y_ref):
    d0_ref = dg_ref
    dinv = lax.rsqrt(d0_ref[...] + d1_ref[...] + 1.0)
    y_ref[...] = jnp.dot(x_ref[...], w_ref[...],
                         preferred_element_type=jnp.float32) * dinv


_tc_prep = pl.pallas_call(
    _tc_prep_body,
    grid=(_GRID,),
    in_specs=[
        pl.BlockSpec((_BN, _D), lambda i: (i, 0)),
        pl.BlockSpec((_D, _D), lambda i: (0, 0)),
        pl.BlockSpec((_BN, 1), lambda i: (i, 0)),
        pl.BlockSpec((_BN, 1), lambda i: (i + _GRID, 0)),
    ],
    out_specs=pl.BlockSpec((_BN, _D), lambda i: (i, 0)),
    out_shape=jax.ShapeDtypeStruct((_NP, _D), jnp.float32),
)


def _tc_mid_body(a0_ref, a1_ref, y_ref, d0_ref, d1_ref, b_ref, w_ref,
                 o_ref):
    dinv = lax.rsqrt(d0_ref[...] + d1_ref[...] + 1.0)
    h = dinv * (a0_ref[...] + a1_ref[...] + y_ref[...]) + b_ref[...]
    h = jnp.maximum(h, 0.0)
    o_ref[...] = jnp.dot(h, w_ref[...],
                         preferred_element_type=jnp.float32) * dinv


_tc_mid = pl.pallas_call(
    _tc_mid_body,
    grid=(_GRID,),
    in_specs=[
        pl.BlockSpec((_BN, _D), lambda i: (i, 0)),
        pl.BlockSpec((_BN, _D), lambda i: (i + _GRID, 0)),
        pl.BlockSpec((_BN, _D), lambda i: (i, 0)),
        pl.BlockSpec((_BN, 1), lambda i: (i, 0)),
        pl.BlockSpec((_BN, 1), lambda i: (i + _GRID, 0)),
        pl.BlockSpec((1, _D), lambda i: (0, 0)),
        pl.BlockSpec((_D, _D), lambda i: (0, 0)),
    ],
    out_specs=pl.BlockSpec((_BN, _D), lambda i: (i, 0)),
    out_shape=jax.ShapeDtypeStruct((_NP, _D), jnp.float32),
)


def _tc_out_body(a0_ref, a1_ref, y_ref, d0_ref, d1_ref, b_ref, o_ref):
    dinv = lax.rsqrt(d0_ref[...] + d1_ref[...] + 1.0)
    h = dinv * (a0_ref[...] + a1_ref[...] + y_ref[...]) + b_ref[...]
    o_ref[...] = jnp.maximum(h, 0.0)


_tc_out = pl.pallas_call(
    _tc_out_body,
    grid=(_GRID,),
    in_specs=[
        pl.BlockSpec((_BN, _D), lambda i: (i, 0)),
        pl.BlockSpec((_BN, _D), lambda i: (i + _GRID, 0)),
        pl.BlockSpec((_BN, _D), lambda i: (i, 0)),
        pl.BlockSpec((_BN, 1), lambda i: (i, 0)),
        pl.BlockSpec((_BN, 1), lambda i: (i + _GRID, 0)),
        pl.BlockSpec((1, _D), lambda i: (0, 0)),
    ],
    out_specs=pl.BlockSpec((_BN, _D), lambda i: (i, 0)),
    out_shape=jax.ShapeDtypeStruct((_NP, _D), jnp.float32),
)


@jax.jit
def kernel(x, edge_index, W1, b1, W2, b2):
    ei = edge_index.reshape(2, _NW, _NCHUNK, _K)
    xp = jnp.pad(x, ((0, _NP - _N), (0, 0)))
    degp = _sc_deg(ei).reshape(_NC * _DEGN, 1)
    y1 = _tc_prep(xp, W1, degp, degp)
    accp1 = _sc_agg(y1, ei)
    y2 = _tc_mid(accp1, accp1, y1, degp, degp, b1.reshape(1, _D), W2)
    accp2 = _sc_agg(y2, ei)
    out = _tc_out(accp2, accp2, y2, degp, degp, b2.reshape(1, _D))
    return out[:_N]
